# initial kernel scaffold (unmeasured)
import jax
import jax.numpy as jnp
from jax import lax
from jax.experimental import pallas as pl
from jax.experimental.pallas import tpu as pltpu

N_DEV = 32


def kernel(x, w_mat):
    m_per, k_dim = x.shape
    _, n = w_mat.shape
    n_per = n // N_DEV

    def body(x_ref, w_ref, out_ref, y_ref, send_sems, recv_sems):
        my = lax.axis_index("i")

        y_ref[...] = jnp.dot(
            x_ref[...], w_ref[...], preferred_element_type=jnp.float32
        )

        out_ref[pl.ds(my * m_per, m_per), :] = y_ref[:, pl.ds(my * n_per, n_per)]

        rdmas = []
        for kk in range(1, N_DEV):
            d = lax.rem(my + kk, N_DEV)
            rdma = pltpu.make_async_remote_copy(
                src_ref=y_ref.at[:, pl.ds(d * n_per, n_per)],
                dst_ref=out_ref.at[pl.ds(my * m_per, m_per), :],
                send_sem=send_sems.at[kk - 1],
                recv_sem=recv_sems.at[kk - 1],
                device_id=(d,),
                device_id_type=pl.DeviceIdType.MESH,
            )
            rdma.start()
            rdmas.append(rdma)
        for rdma in rdmas:
            rdma.wait()

    return pl.pallas_call(
        body,
        out_shape=jax.ShapeDtypeStruct((N_DEV * m_per, n_per), jnp.float32),
        in_specs=[
            pl.BlockSpec(memory_space=pltpu.VMEM),
            pl.BlockSpec(memory_space=pltpu.VMEM),
        ],
        out_specs=pl.BlockSpec(memory_space=pltpu.VMEM),
        scratch_shapes=[
            pltpu.VMEM((m_per, n), jnp.float32),
            pltpu.SemaphoreType.DMA((N_DEV - 1,)),
            pltpu.SemaphoreType.DMA((N_DEV - 1,)),
        ],
    )(x, w_mat)


# baseline (device time: 31223 ns/iter reference)
import jax
import jax.numpy as jnp
from jax import lax
from jax.experimental import pallas as pl
from jax.experimental.pallas import tpu as pltpu

N_DEV = 32


def kernel(x, w_mat):
    m_per, k_dim = x.shape
    _, n = w_mat.shape
    n_per = n // N_DEV

    w_blocks = w_mat.reshape(k_dim, N_DEV, n_per).transpose(1, 0, 2)

    def body(x_ref, w_ref, out_ref, y_ref, send_sems, recv_sems):
        my = lax.axis_index("i")

        for d in range(N_DEV):
            y_ref[d] = jnp.dot(
                x_ref[...], w_ref[d], preferred_element_type=jnp.float32
            )

        out_ref[pl.ds(my * m_per, m_per), :] = y_ref[my]

        rdmas = []
        for kk in range(1, N_DEV):
            d = lax.rem(my + kk, N_DEV)
            rdma = pltpu.make_async_remote_copy(
                src_ref=y_ref.at[d],
                dst_ref=out_ref.at[pl.ds(my * m_per, m_per), :],
                send_sem=send_sems.at[kk - 1],
                recv_sem=recv_sems.at[kk - 1],
                device_id=(d,),
                device_id_type=pl.DeviceIdType.MESH,
            )
            rdma.start()
            rdmas.append(rdma)
        for rdma in rdmas:
            rdma.wait()

    return pl.pallas_call(
        body,
        out_shape=jax.ShapeDtypeStruct((N_DEV * m_per, n_per), jnp.float32),
        in_specs=[
            pl.BlockSpec(memory_space=pltpu.VMEM),
            pl.BlockSpec(memory_space=pltpu.VMEM),
        ],
        out_specs=pl.BlockSpec(memory_space=pltpu.VMEM),
        scratch_shapes=[
            pltpu.VMEM((N_DEV, m_per, n_per), jnp.float32),
            pltpu.SemaphoreType.DMA((N_DEV - 1,)),
            pltpu.SemaphoreType.DMA((N_DEV - 1,)),
        ],
    )(x, w_blocks)


# device time: 19396 ns/iter; 1.6098x vs baseline; 1.6098x over previous
import jax
import jax.numpy as jnp
from jax import lax
from jax.experimental import pallas as pl
from jax.experimental.pallas import tpu as pltpu

N_DEV = 32


def kernel(x, w_mat):
    m_per, k_dim = x.shape
    _, n = w_mat.shape
    n_per = n // N_DEV

    def body(x_ref, w_ref, out_ref, y_ref, send_sems, recv_sems):
        my = lax.axis_index("i")

        barrier_sem = pltpu.get_barrier_semaphore()
        for p in range(1, N_DEV):
            pl.semaphore_signal(
                barrier_sem,
                inc=1,
                device_id=(lax.rem(my + p, N_DEV),),
                device_id_type=pl.DeviceIdType.MESH,
            )

        y = jnp.dot(x_ref[...], w_ref[...], preferred_element_type=jnp.float32)
        for d in range(N_DEV):
            y_ref[d] = y[:, d * n_per:(d + 1) * n_per]

        out_ref[pl.ds(my * m_per, m_per), :] = y_ref[my]

        pl.semaphore_wait(barrier_sem, N_DEV - 1)

        rdmas = []
        for kk in range(1, N_DEV):
            d = lax.rem(my + kk, N_DEV)
            rdma = pltpu.make_async_remote_copy(
                src_ref=y_ref.at[d],
                dst_ref=out_ref.at[pl.ds(my * m_per, m_per), :],
                send_sem=send_sems.at[kk - 1],
                recv_sem=recv_sems.at[kk - 1],
                device_id=(d,),
                device_id_type=pl.DeviceIdType.MESH,
            )
            rdma.start()
            rdmas.append(rdma)
        for rdma in rdmas:
            rdma.wait()

    return pl.pallas_call(
        body,
        out_shape=jax.ShapeDtypeStruct((N_DEV * m_per, n_per), jnp.float32),
        in_specs=[
            pl.BlockSpec(memory_space=pltpu.VMEM),
            pl.BlockSpec(memory_space=pltpu.VMEM),
        ],
        out_specs=pl.BlockSpec(memory_space=pltpu.VMEM),
        scratch_shapes=[
            pltpu.VMEM((N_DEV, m_per, n_per), jnp.float32),
            pltpu.SemaphoreType.DMA((N_DEV - 1,)),
            pltpu.SemaphoreType.DMA((N_DEV - 1,)),
        ],
        compiler_params=pltpu.CompilerParams(collective_id=0),
    )(x, w_mat)


# device time: 16911 ns/iter; 1.8463x vs baseline; 1.1469x over previous
import os

import jax
import jax.numpy as jnp
from jax import lax
from jax.experimental import pallas as pl
from jax.experimental.pallas import tpu as pltpu

N_DEV = 32
_VARIANT = os.environ.get("KV", "full")


def kernel(x, w_mat):
    m_per, k_dim = x.shape
    _, n = w_mat.shape
    n_per = n // N_DEV

    def body(x_ref, w_ref, out_ref, y_ref, send_sems, recv_sems):
        my = lax.axis_index("i")

        if _VARIANT != "nocomm":
            barrier_sem = pltpu.get_barrier_semaphore()
            n_bar = 2 if _VARIANT == "weakbar" else N_DEV - 1
            peers = [1, N_DEV - 1] if _VARIANT == "weakbar" else range(1, N_DEV)
            for p in peers:
                pl.semaphore_signal(
                    barrier_sem,
                    inc=1,
                    device_id=(lax.rem(my + p, N_DEV),),
                    device_id_type=pl.DeviceIdType.MESH,
                )

        if _VARIANT == "nogemm":
            for d in range(N_DEV):
                y_ref[d] = jnp.zeros((m_per, n_per), jnp.float32)
        else:
            y = jnp.dot(
                x_ref[...], w_ref[...], preferred_element_type=jnp.float32
            )
            for d in range(N_DEV):
                y_ref[d] = y[:, d * n_per:(d + 1) * n_per]

        out_ref[pl.ds(my * m_per, m_per), :] = y_ref[my]

        if _VARIANT == "nocomm":
            return

        pl.semaphore_wait(barrier_sem, n_bar)

        rdmas = []
        for kk in range(1, N_DEV):
            d = lax.rem(my + kk, N_DEV)
            rdma = pltpu.make_async_remote_copy(
                src_ref=y_ref.at[d],
                dst_ref=out_ref.at[pl.ds(my * m_per, m_per), :],
                send_sem=send_sems.at[kk - 1],
                recv_sem=recv_sems.at[kk - 1],
                device_id=(d,),
                device_id_type=pl.DeviceIdType.MESH,
            )
            rdma.start()
            rdmas.append(rdma)
        for rdma in rdmas:
            rdma.wait()

    return pl.pallas_call(
        body,
        out_shape=jax.ShapeDtypeStruct((N_DEV * m_per, n_per), jnp.float32),
        in_specs=[
            pl.BlockSpec(memory_space=pltpu.VMEM),
            pl.BlockSpec(memory_space=pltpu.VMEM),
        ],
        out_specs=pl.BlockSpec(memory_space=pltpu.VMEM),
        scratch_shapes=[
            pltpu.VMEM((N_DEV, m_per, n_per), jnp.float32),
            pltpu.SemaphoreType.DMA((N_DEV - 1,)),
            pltpu.SemaphoreType.DMA((N_DEV - 1,)),
        ],
        compiler_params=(
            None
            if _VARIANT == "nocomm"
            else pltpu.CompilerParams(collective_id=0)
        ),
    )(x, w_mat)
